# baseline (device time: 41928 ns/iter reference)
import jax
import jax.numpy as jnp
from jax import lax
from jax.experimental import pallas as pl
from jax.experimental.pallas import tpu as pltpu

NC = 2


def kernel(x, Win0, Wout0, Win1, Wout1, Win2, Wout2):
    b, d_y = x.shape
    _, h_x = Win0.shape
    ch = b // NC
    bf16 = jnp.bfloat16

    def body(x_ref, win0_ref, wout0_ref, win1_ref, wout1_ref, win2_ref,
             wout2_ref, out_ref,
             sendy_ref, recvy_ref, sendx_ref, recvx_ref,
             ysend_sems, yrecv_sems, xsend_sems, xrecv_sems):
        my_x = lax.axis_index("x")
        my_y = lax.axis_index("y")
        y_peer = (my_x, 1 - my_y)
        x_peer = (1 - my_x, my_y)

        wins = [win0_ref, win1_ref, win2_ref]
        wouts = [wout0_ref, wout1_ref, wout2_ref]
        started = []

        def rdma_y(l, c):
            return pltpu.make_async_remote_copy(
                src_ref=sendy_ref.at[l, c],
                dst_ref=recvy_ref.at[l, c],
                send_sem=ysend_sems.at[l, c],
                recv_sem=yrecv_sems.at[l, c],
                device_id=y_peer,
                device_id_type=pl.DeviceIdType.MESH,
            )

        def rdma_x(l, c):
            return pltpu.make_async_remote_copy(
                src_ref=sendx_ref.at[l, c],
                dst_ref=recvx_ref.at[l, c],
                send_sem=xsend_sems.at[l, c],
                recv_sem=xrecv_sems.at[l, c],
                device_id=x_peer,
                device_id_type=pl.DeviceIdType.MESH,
            )

        win_bf = wins[0][...].astype(bf16)
        x_bf = x_ref[...].astype(bf16)
        p1 = jnp.dot(x_bf, win_bf, preferred_element_type=jnp.float32)
        for c in range(NC):
            sendy_ref[0, c] = p1[c * ch:(c + 1) * ch, :].astype(bf16)
            r = rdma_y(0, c)
            r.start()
            started.append(r)

        for l in range(3):
            wout_bf = wouts[l][...].astype(bf16)
            win_next_bf = wins[l + 1][...].astype(bf16) if l < 2 else None

            for c in range(NC):
                rdma_y(l, c).wait_recv()
                h_c = jnp.maximum(sendy_ref[l, c] + recvy_ref[l, c], 0.0)
                p2_c = jnp.dot(h_c, wout_bf, preferred_element_type=jnp.float32)
                sendx_ref[l, c] = p2_c.astype(bf16)
                r = rdma_x(l, c)
                r.start()
                started.append(r)

            for c in range(NC):
                rdma_x(l, c).wait_recv()
                if l < 2:
                    x_c = sendx_ref[l, c] + recvx_ref[l, c]
                    p1_c = jnp.dot(x_c, win_next_bf,
                                   preferred_element_type=jnp.float32)
                    sendy_ref[l + 1, c] = p1_c.astype(bf16)
                    r = rdma_y(l + 1, c)
                    r.start()
                    started.append(r)
                else:
                    out_ref[c * ch:(c + 1) * ch, :] = (
                        sendx_ref[l, c].astype(jnp.float32)
                        + recvx_ref[l, c].astype(jnp.float32)
                    )

        for r in started:
            r.wait_send()

    return pl.pallas_call(
        body,
        out_shape=jax.ShapeDtypeStruct((b, d_y), jnp.float32),
        in_specs=[pl.BlockSpec(memory_space=pltpu.VMEM)] * 7,
        out_specs=pl.BlockSpec(memory_space=pltpu.VMEM),
        scratch_shapes=[
            pltpu.VMEM((3, NC, ch, h_x), bf16),
            pltpu.VMEM((3, NC, ch, h_x), bf16),
            pltpu.VMEM((3, NC, ch, d_y), bf16),
            pltpu.VMEM((3, NC, ch, d_y), bf16),
            pltpu.SemaphoreType.DMA((3, NC)),
            pltpu.SemaphoreType.DMA((3, NC)),
            pltpu.SemaphoreType.DMA((3, NC)),
            pltpu.SemaphoreType.DMA((3, NC)),
        ],
    )(x, Win0, Wout0, Win1, Wout1, Win2, Wout2)


# device time: 40571 ns/iter; 1.0334x vs baseline; 1.0334x over previous
import jax
import jax.numpy as jnp
from jax import lax
from jax.experimental import pallas as pl
from jax.experimental.pallas import tpu as pltpu

NC = 4


def kernel(x, Win0, Wout0, Win1, Wout1, Win2, Wout2):
    b, d_y = x.shape
    _, h_x = Win0.shape
    ch = b // NC
    bf16 = jnp.bfloat16

    def body(x_ref, win0_ref, wout0_ref, win1_ref, wout1_ref, win2_ref,
             wout2_ref, out_ref,
             sendy_ref, recvy_ref, sendx_ref, recvx_ref,
             ysend_sems, yrecv_sems, xsend_sems, xrecv_sems):
        my_x = lax.axis_index("x")
        my_y = lax.axis_index("y")
        y_peer = (my_x, 1 - my_y)
        x_peer = (1 - my_x, my_y)

        wins = [win0_ref, win1_ref, win2_ref]
        wouts = [wout0_ref, wout1_ref, wout2_ref]
        started = []

        def rdma_y(l, c):
            return pltpu.make_async_remote_copy(
                src_ref=sendy_ref.at[l, c],
                dst_ref=recvy_ref.at[l, c],
                send_sem=ysend_sems.at[l, c],
                recv_sem=yrecv_sems.at[l, c],
                device_id=y_peer,
                device_id_type=pl.DeviceIdType.MESH,
            )

        def rdma_x(l, c):
            return pltpu.make_async_remote_copy(
                src_ref=sendx_ref.at[l, c],
                dst_ref=recvx_ref.at[l, c],
                send_sem=xsend_sems.at[l, c],
                recv_sem=xrecv_sems.at[l, c],
                device_id=x_peer,
                device_id_type=pl.DeviceIdType.MESH,
            )

        win_bf = wins[0][...].astype(bf16)
        x_bf = x_ref[...].astype(bf16)
        p1 = jnp.dot(x_bf, win_bf, preferred_element_type=jnp.float32)
        for c in range(NC):
            sendy_ref[0, c] = p1[c * ch:(c + 1) * ch, :].astype(bf16)
            r = rdma_y(0, c)
            r.start()
            started.append(r)

        for l in range(3):
            wout_bf = wouts[l][...].astype(bf16)
            win_next_bf = wins[l + 1][...].astype(bf16) if l < 2 else None

            for c in range(NC):
                rdma_y(l, c).wait_recv()
                h_c = jnp.maximum(sendy_ref[l, c] + recvy_ref[l, c], 0.0)
                p2_c = jnp.dot(h_c, wout_bf, preferred_element_type=jnp.float32)
                sendx_ref[l, c] = p2_c.astype(bf16)
                r = rdma_x(l, c)
                r.start()
                started.append(r)

            for c in range(NC):
                rdma_x(l, c).wait_recv()
                if l < 2:
                    x_c = sendx_ref[l, c] + recvx_ref[l, c]
                    p1_c = jnp.dot(x_c, win_next_bf,
                                   preferred_element_type=jnp.float32)
                    sendy_ref[l + 1, c] = p1_c.astype(bf16)
                    r = rdma_y(l + 1, c)
                    r.start()
                    started.append(r)
                else:
                    out_ref[c * ch:(c + 1) * ch, :] = (
                        sendx_ref[l, c].astype(jnp.float32)
                        + recvx_ref[l, c].astype(jnp.float32)
                    )

        for r in started:
            r.wait_send()

    return pl.pallas_call(
        body,
        out_shape=jax.ShapeDtypeStruct((b, d_y), jnp.float32),
        in_specs=[pl.BlockSpec(memory_space=pltpu.VMEM)] * 7,
        out_specs=pl.BlockSpec(memory_space=pltpu.VMEM),
        scratch_shapes=[
            pltpu.VMEM((3, NC, ch, h_x), bf16),
            pltpu.VMEM((3, NC, ch, h_x), bf16),
            pltpu.VMEM((3, NC, ch, d_y), bf16),
            pltpu.VMEM((3, NC, ch, d_y), bf16),
            pltpu.SemaphoreType.DMA((3, NC)),
            pltpu.SemaphoreType.DMA((3, NC)),
            pltpu.SemaphoreType.DMA((3, NC)),
            pltpu.SemaphoreType.DMA((3, NC)),
        ],
    )(x, Win0, Wout0, Win1, Wout1, Win2, Wout2)


# device time: 38682 ns/iter; 1.0839x vs baseline; 1.0488x over previous
import jax
import jax.numpy as jnp
from jax import lax
from jax.experimental import pallas as pl
from jax.experimental.pallas import tpu as pltpu

NC = 2


def kernel(x, Win0, Wout0, Win1, Wout1, Win2, Wout2):
    b, d_y = x.shape
    _, h_x = Win0.shape
    ch = b // NC
    bf16 = jnp.bfloat16

    def body(x_ref, win0_ref, wout0_ref, win1_ref, wout1_ref, win2_ref,
             wout2_ref, out_ref,
             sendy_ref, recvy_ref, sendx_ref, recvx_ref,
             ysend_sems, yrecv_sems, xsend_sems, xrecv_sems):
        my_x = lax.axis_index("x")
        my_y = lax.axis_index("y")
        y_peer = (my_x, 1 - my_y)
        x_peer = (1 - my_x, my_y)

        wins = [win0_ref, win1_ref, win2_ref]
        wouts = [wout0_ref, wout1_ref, wout2_ref]
        started = []

        def rdma_y(l, c):
            return pltpu.make_async_remote_copy(
                src_ref=sendy_ref.at[l, c],
                dst_ref=recvy_ref.at[l, c],
                send_sem=ysend_sems.at[l, c],
                recv_sem=yrecv_sems.at[l, c],
                device_id=y_peer,
                device_id_type=pl.DeviceIdType.MESH,
            )

        def rdma_x(l, c):
            return pltpu.make_async_remote_copy(
                src_ref=sendx_ref.at[l, c],
                dst_ref=recvx_ref.at[l, c],
                send_sem=xsend_sems.at[l, c],
                recv_sem=xrecv_sems.at[l, c],
                device_id=x_peer,
                device_id_type=pl.DeviceIdType.MESH,
            )

        barrier_sem = pltpu.get_barrier_semaphore()
        for nbr in [y_peer, x_peer]:
            pl.semaphore_signal(barrier_sem, inc=1, device_id=nbr,
                                device_id_type=pl.DeviceIdType.MESH)
        pl.semaphore_wait(barrier_sem, 2)

        win_bf = wins[0][...].astype(bf16)
        for c in range(NC):
            x_c = x_ref[c * ch:(c + 1) * ch, :].astype(bf16)
            p1_c = jnp.dot(x_c, win_bf, preferred_element_type=jnp.float32)
            sendy_ref[0, c] = p1_c.astype(bf16)
            r = rdma_y(0, c)
            r.start()
            started.append(r)

        for l in range(3):
            wout_bf = wouts[l][...].astype(bf16)
            win_next_bf = wins[l + 1][...].astype(bf16) if l < 2 else None

            for c in range(NC):
                rdma_y(l, c).wait_recv()
                h_c = jnp.maximum(sendy_ref[l, c] + recvy_ref[l, c], 0.0)
                p2_c = jnp.dot(h_c, wout_bf, preferred_element_type=jnp.float32)
                sendx_ref[l, c] = p2_c.astype(bf16)
                r = rdma_x(l, c)
                r.start()
                started.append(r)

            for c in range(NC):
                rdma_x(l, c).wait_recv()
                if l < 2:
                    x_c = sendx_ref[l, c] + recvx_ref[l, c]
                    p1_c = jnp.dot(x_c, win_next_bf,
                                   preferred_element_type=jnp.float32)
                    sendy_ref[l + 1, c] = p1_c.astype(bf16)
                    r = rdma_y(l + 1, c)
                    r.start()
                    started.append(r)
                else:
                    out_ref[c * ch:(c + 1) * ch, :] = (
                        sendx_ref[l, c].astype(jnp.float32)
                        + recvx_ref[l, c].astype(jnp.float32)
                    )

        for r in started:
            r.wait_send()

    return pl.pallas_call(
        body,
        out_shape=jax.ShapeDtypeStruct((b, d_y), jnp.float32),
        in_specs=[pl.BlockSpec(memory_space=pltpu.VMEM)] * 7,
        out_specs=pl.BlockSpec(memory_space=pltpu.VMEM),
        scratch_shapes=[
            pltpu.VMEM((3, NC, ch, h_x), bf16),
            pltpu.VMEM((3, NC, ch, h_x), bf16),
            pltpu.VMEM((3, NC, ch, d_y), bf16),
            pltpu.VMEM((3, NC, ch, d_y), bf16),
            pltpu.SemaphoreType.DMA((3, NC)),
            pltpu.SemaphoreType.DMA((3, NC)),
            pltpu.SemaphoreType.DMA((3, NC)),
            pltpu.SemaphoreType.DMA((3, NC)),
        ],
        compiler_params=pltpu.CompilerParams(collective_id=0),
    )(x, Win0, Wout0, Win1, Wout1, Win2, Wout2)
